# baseline (speedup 1.0000x reference)
import functools

import jax
import jax.numpy as jnp
from jax import lax
from jax.experimental import pallas as pl
from jax.experimental.pallas import tpu as pltpu
from jax.experimental.pallas import tpu_sc as plsc

N = 10000
E = 320000
D = 128
EDGE_DIM = 16
N_ACTIVE = 4096

NC, NS = 2, 16
NW = NC * NS
C = 128
E_PAD = 327680
EPW = E_PAD // NW
NCH = EPW // C
NAGG = 10112
ZR = NAGG // NS

def _sc_mesh():
    return plsc.VectorSubcoreMesh(
        core_axis_name="c", subcore_axis_name="s",
        num_cores=NC, num_subcores=NS)


def _worker_id():
    return lax.axis_index("s") * NC + lax.axis_index("c")


def _relu_rows(dst_ref, *src_refs, rows):
    def row(r, _):
        for cg in range(D // 16):
            sl = pl.ds(cg * 16, 16)
            acc = src_refs[0][r, sl]
            for s in src_refs[1:]:
                acc = acc + s[r, sl]
            dst_ref[r, sl] = jnp.maximum(acc, 0.0)
        return 0
    lax.fori_loop(0, rows, row, 0)


CM = 80
NCHM = EPW // CM
SLAB = 16


@functools.cache
def _sc_message_fn():
    @functools.partial(
        pl.kernel,
        out_type=jax.ShapeDtypeStruct((NC, NAGG, D), jnp.float32),
        mesh=_sc_mesh(),
        scratch_types=[
            pltpu.VMEM((SLAB, CM), jnp.int32),
            pltpu.VMEM((SLAB, CM), jnp.int32),
            pltpu.VMEM((2, CM, D), jnp.float32),
            pltpu.VMEM((2, CM, D), jnp.float32),
            pltpu.VMEM_SHARED((NAGG, D), jnp.float32),
            pltpu.SemaphoreType.DMA,
            pltpu.SemaphoreType.DMA,
            pltpu.SemaphoreType.DMA,
            pltpu.SemaphoreType.DMA,
        ],
    )
    def _sc_message(hm, em, srcp, dstp, zeros, out, src_v, dst_v, em_v, g_v,
                    agg, se0, se1, sg0, sg1):
        c = lax.axis_index("c")
        s = lax.axis_index("s")
        w = _worker_id()
        se = (se0, se1)
        sg = (sg0, sg1)
        pltpu.sync_copy(zeros.at[pl.ds(s * ZR, ZR)], agg.at[pl.ds(s * ZR, ZR)])
        plsc.subcore_barrier()

        def load_slab(i):
            i = pl.multiple_of(i, SLAB)
            pltpu.sync_copy(srcp.at[w, pl.ds(i, SLAB)], src_v)
            pltpu.sync_copy(dstp.at[w, pl.ds(i, SLAB)], dst_v)

        def start(i, b):
            pltpu.async_copy(em.at[pl.ds(w * EPW + i * CM, CM)], em_v.at[b],
                             se[b])
            pltpu.async_copy(hm.at[src_v.at[i % SLAB]], g_v.at[b], sg[b])

        n_my = jnp.minimum(NCHM, E // CM - w * NCHM)

        load_slab(0)
        start(0, 0)

        def pair(j, _):
            for b in (0, 1):
                i = 2 * j + b
                nxt = i + 1

                @pl.when((nxt < n_my) & (nxt % SLAB != 0))
                def _():
                    start(nxt, 1 - b)

                pltpu.make_async_copy(em.at[pl.ds(0, CM)], em_v.at[b],
                                      se[b]).wait()
                pltpu.make_async_copy(hm.at[pl.ds(0, CM)], g_v.at[b],
                                      sg[b]).wait()
                _relu_rows(g_v.at[b], g_v.at[b], em_v.at[b], rows=CM)
                pltpu.sync_copy(g_v.at[b], agg.at[dst_v.at[i % SLAB]],
                                add=True)

                @pl.when((nxt < n_my) & (nxt % SLAB == 0))
                def _():
                    load_slab(nxt)
                    start(nxt, 1 - b)
            return 0

        lax.fori_loop(0, n_my // 2, pair, 0)
        plsc.subcore_barrier()
        pltpu.sync_copy(agg.at[pl.ds(s * ZR, ZR)],
                        out.at[c, pl.ds(s * ZR, ZR)])

    return _sc_message


@functools.cache
def _sc_edge_update_fn():
    @functools.partial(
        pl.kernel,
        out_type=jax.ShapeDtypeStruct((E_PAD, D), jnp.float32),
        mesh=_sc_mesh(),
        scratch_types=[
            pltpu.VMEM((NCH, C), jnp.int32),
            pltpu.VMEM((NCH, C), jnp.int32),
            pltpu.VMEM((2, C, D), jnp.float32),
            pltpu.VMEM((2, C, D), jnp.float32),
            pltpu.VMEM((2, C, D), jnp.float32),
            pltpu.SemaphoreType.DMA,
            pltpu.SemaphoreType.DMA,
            pltpu.SemaphoreType.DMA,
            pltpu.SemaphoreType.DMA,
            pltpu.SemaphoreType.DMA,
            pltpu.SemaphoreType.DMA,
        ],
    )
    def _sc_edge_update(hu1, hu2, eu, srcp, dstp, out, src_v, dst_v, g1, g2,
                        ev, sv0, sv1, sa0, sa1, sb0, sb1):
        w = _worker_id()
        sv = (sv0, sv1)
        sa = (sa0, sa1)
        sb = (sb0, sb1)
        pltpu.sync_copy(srcp.at[w], src_v)
        pltpu.sync_copy(dstp.at[w], dst_v)
        n_my = jnp.minimum(NCH, E // C - w * NCH)

        def start(i, b):
            pltpu.async_copy(eu.at[pl.ds(w * EPW + i * C, C)], ev.at[b],
                             sv[b])
            pltpu.async_copy(hu1.at[src_v.at[i]], g1.at[b], sa[b])
            pltpu.async_copy(hu2.at[dst_v.at[i]], g2.at[b], sb[b])

        start(0, 0)

        def pair(j, _):
            for b in (0, 1):
                i = 2 * j + b
                nxt = i + 1

                @pl.when(nxt < n_my)
                def _():
                    start(nxt, 1 - b)

                pltpu.make_async_copy(eu.at[pl.ds(0, C)], ev.at[b],
                                      sv[b]).wait()
                pltpu.make_async_copy(hu1.at[pl.ds(0, C)], g1.at[b],
                                      sa[b]).wait()
                pltpu.make_async_copy(hu2.at[pl.ds(0, C)], g2.at[b],
                                      sb[b]).wait()
                _relu_rows(ev.at[b], g1.at[b], g2.at[b], ev.at[b], rows=C)
                pltpu.sync_copy(ev.at[b], out.at[pl.ds(w * EPW + i * C, C)])
            return 0

        lax.fori_loop(0, n_my // 2, pair, 0)

    return _sc_edge_update


_APW = N_ACTIVE // NW


@functools.cache
def _sc_active_gather_fn():
    @functools.partial(
        pl.kernel,
        out_type=jax.ShapeDtypeStruct((N_ACTIVE, D), jnp.float32),
        mesh=_sc_mesh(),
        scratch_types=[
            pltpu.VMEM((_APW,), jnp.int32),
            pltpu.VMEM((_APW, D), jnp.float32),
            pltpu.SemaphoreType.DMA,
        ],
    )
    def _sc_active_gather(h, nid, out, idx_v, rows_v, sem):
        w = _worker_id()
        base = w * _APW
        pltpu.sync_copy(nid.at[pl.ds(base, _APW)], idx_v)
        pltpu.async_copy(h.at[idx_v], rows_v, sem).wait()
        pltpu.sync_copy(rows_v, out.at[pl.ds(base, _APW)])

    return _sc_active_gather


def _dot(a, b):
    return jnp.dot(a, b, preferred_element_type=jnp.float32)


def _encode_nodes_body(x, wne, bne, wmh, wself, bn, hm_o, hs_o):
    h = jnp.maximum(_dot(x[...], wne[...]) + bne[...], 0.0)
    hm_o[...] = _dot(h, wmh[...])
    hs_o[...] = _dot(h, wself[...]) + bn[...]


def _encode_nodes(x, wne, bne, wmh, wself, bn):
    return pl.pallas_call(
        _encode_nodes_body,
        out_shape=[jax.ShapeDtypeStruct((N, D), jnp.float32)] * 2,
    )(x, wne, bne, wmh, wself, bn)


_BE = 4000
_GE = E // _BE


def _encode_edges_body(ea, wee, bee, wme, e_o, em_o):
    e = jnp.maximum(_dot(ea[...], wee[...]) + bee[...], 0.0)
    e_o[...] = e
    em_o[...] = _dot(e, wme[...])


def _encode_edges(ea, wee, bee, wme):
    blk = lambda r, c: pl.BlockSpec((r, c), lambda i: (0, 0))
    return pl.pallas_call(
        _encode_edges_body,
        grid=(_GE,),
        in_specs=[pl.BlockSpec((_BE, EDGE_DIM), lambda i: (i, 0)),
                  blk(EDGE_DIM, D), blk(1, D), blk(D, D)],
        out_specs=[pl.BlockSpec((_BE, D), lambda i: (i, 0))] * 2,
        out_shape=[jax.ShapeDtypeStruct((E_PAD, D), jnp.float32)] * 2,
    )(ea, wee, bee, wme)


def _edge_mm1_body(e, wme, em_o):
    em_o[...] = _dot(e[...], wme[...])


def _edge_mm1(e, wme):
    return pl.pallas_call(
        _edge_mm1_body,
        grid=(_GE,),
        in_specs=[pl.BlockSpec((_BE, D), lambda i: (i, 0)),
                  pl.BlockSpec((D, D), lambda i: (0, 0))],
        out_specs=pl.BlockSpec((_BE, D), lambda i: (i, 0)),
        out_shape=jax.ShapeDtypeStruct((E_PAD, D), jnp.float32),
    )(e, wme)


def _eu_mm_body(e, wue, be, eu_o):
    eu_o[...] = _dot(e[...], wue[...]) + be[...]


def _eu_mm(e, wue, be):
    blk = lambda r, c: pl.BlockSpec((r, c), lambda i: (0, 0))
    return pl.pallas_call(
        _eu_mm_body,
        grid=(_GE,),
        in_specs=[pl.BlockSpec((_BE, D), lambda i: (i, 0)),
                  blk(D, D), blk(1, D)],
        out_specs=pl.BlockSpec((_BE, D), lambda i: (i, 0)),
        out_shape=jax.ShapeDtypeStruct((E_PAD, D), jnp.float32),
    )(e, wue, be)


def _node_update_mid_body(hs, agg, wmh, wself, bn, wes, wed,
                          hm_o, hs_o, hu1_o, hu2_o):
    a = agg[...]
    h = jnp.maximum(hs[...] + a[0, :N, :] + a[1, :N, :], 0.0)
    hm_o[...] = _dot(h, wmh[...])
    hs_o[...] = _dot(h, wself[...]) + bn[...]
    hu1_o[...] = _dot(h, wes[...])
    hu2_o[...] = _dot(h, wed[...])


def _node_update_mid(hs, agg, wmh, wself, bn, wes, wed):
    return pl.pallas_call(
        _node_update_mid_body,
        out_shape=[jax.ShapeDtypeStruct((N, D), jnp.float32)] * 4,
    )(hs, agg, wmh, wself, bn, wes, wed)


def _node_update_last_body(hs, agg, h_o):
    a = agg[...]
    h_o[...] = jnp.maximum(hs[...] + a[0, :N, :] + a[1, :N, :], 0.0)


def _node_update_last(hs, agg):
    return pl.pallas_call(
        _node_update_last_body,
        out_shape=jax.ShapeDtypeStruct((N, D), jnp.float32),
    )(hs, agg)


def _mlp_body(ha, w1, b1, w2, b2, out):
    z = jnp.maximum(_dot(ha[...], w1[...]) + b1[...], 0.0)
    out[...] = _dot(z, w2[...]) + b2[...]


def _mlp(ha, w1, b1, w2, b2):
    return pl.pallas_call(
        _mlp_body,
        out_shape=jax.ShapeDtypeStruct((N_ACTIVE, 1), jnp.float32),
    )(ha, w1, b1, w2, b2)


def kernel(x, edge_attr, edge_index, active_nid, W_ne, b_ne, W_ee, b_ee,
           W_msg, W_self, b_n, W_eu, b_e, W1, b1, W2, b2):
    f32 = jnp.float32
    src = edge_index[0].astype(jnp.int32)
    dst = edge_index[1].astype(jnp.int32)
    npad = E_PAD - E
    pad_i = jnp.arange(npad, dtype=jnp.int32)
    src_p = jnp.concatenate([src, (pad_i * 97) % N])
    dst_p = jnp.concatenate([dst, N + pad_i % (NAGG - N)])
    src_pm = src_p.reshape(NW, NCHM, CM)
    dst_pm = dst_p.reshape(NW, NCHM, CM)
    src_pe = src_p.reshape(NW, NCH, C)
    dst_pe = dst_p.reshape(NW, NCH, C)
    zeros_agg = jnp.zeros((NAGG, D), f32)

    Wm_h, Wm_e = W_msg[:D], W_msg[D:]
    We_s, We_d, We_e = W_eu[:D], W_eu[D:2 * D], W_eu[2 * D:]
    row = lambda b: b.reshape(1, -1)

    hm, hs = _encode_nodes(x, W_ne, row(b_ne), Wm_h, W_self, row(b_n))
    e, em = _encode_edges(edge_attr, W_ee, row(b_ee), Wm_e)

    for layer in range(3):
        agg = _sc_message_fn()(hm, em, src_pm, dst_pm, zeros_agg)
        if layer < 2:
            eu = _eu_mm(e, We_e, row(b_e))
            hm, hs, hu1, hu2 = _node_update_mid(
                hs, agg, Wm_h, W_self, row(b_n), We_s, We_d)
            e = _sc_edge_update_fn()(hu1, hu2, eu, src_pe, dst_pe)
            em = _edge_mm1(e, Wm_e)
        else:
            h_fin = _node_update_last(hs, agg)

    ha = _sc_active_gather_fn()(h_fin, active_nid)
    logits = _mlp(ha, W1, row(b1), W2, b2.reshape(1, 1))
    return (logits, active_nid)


# speedup vs baseline: 1.0052x; 1.0052x over previous
import functools

import jax
import jax.numpy as jnp
from jax import lax
from jax.experimental import pallas as pl
from jax.experimental.pallas import tpu as pltpu
from jax.experimental.pallas import tpu_sc as plsc

N = 10000
E = 320000
D = 128
EDGE_DIM = 16
N_ACTIVE = 4096

NC, NS = 2, 16
NW = NC * NS
C = 128
E_PAD = 327680
EPW = E_PAD // NW
NCH = EPW // C
NAGG = 10112
ZR = NAGG // NS

def _sc_mesh():
    return plsc.VectorSubcoreMesh(
        core_axis_name="c", subcore_axis_name="s",
        num_cores=NC, num_subcores=NS)


def _worker_id():
    return lax.axis_index("s") * NC + lax.axis_index("c")


def _relu_rows(dst_ref, *src_refs, rows):
    def row(r, _):
        for cg in range(D // 16):
            sl = pl.ds(cg * 16, 16)
            acc = src_refs[0][r, sl]
            for s in src_refs[1:]:
                acc = acc + s[r, sl]
            dst_ref[r, sl] = jnp.maximum(acc, 0.0)
        return 0
    lax.fori_loop(0, rows, row, 0)


CM = 80
NCHM = EPW // CM
SLAB = 16


@functools.cache
def _sc_message_fn():
    @functools.partial(
        pl.kernel,
        out_type=jax.ShapeDtypeStruct((NC, NAGG, D), jnp.float32),
        mesh=_sc_mesh(),
        scratch_types=[
            pltpu.VMEM((SLAB, CM), jnp.int32),
            pltpu.VMEM((SLAB, CM), jnp.int32),
            pltpu.VMEM((2, CM, D), jnp.float32),
            pltpu.VMEM((2, CM, D), jnp.float32),
            pltpu.VMEM_SHARED((NAGG, D), jnp.float32),
            pltpu.SemaphoreType.DMA,
            pltpu.SemaphoreType.DMA,
            pltpu.SemaphoreType.DMA,
            pltpu.SemaphoreType.DMA,
            pltpu.SemaphoreType.DMA,
            pltpu.SemaphoreType.DMA,
        ],
    )
    def _sc_message(hm, em, srcp, dstp, zeros, out, src_v, dst_v, em_v, g_v,
                    agg, se0, se1, sg0, sg1, ss0, ss1):
        c = lax.axis_index("c")
        s = lax.axis_index("s")
        w = _worker_id()
        se = (se0, se1)
        sg = (sg0, sg1)
        ss = (ss0, ss1)
        pltpu.sync_copy(zeros.at[pl.ds(s * ZR, ZR)], agg.at[pl.ds(s * ZR, ZR)])
        plsc.subcore_barrier()

        def load_slab(i):
            i = pl.multiple_of(i, SLAB)
            pltpu.sync_copy(srcp.at[w, pl.ds(i, SLAB)], src_v)
            pltpu.sync_copy(dstp.at[w, pl.ds(i, SLAB)], dst_v)

        def start(i, b):
            pltpu.async_copy(em.at[pl.ds(w * EPW + i * CM, CM)], em_v.at[b],
                             se[b])
            pltpu.async_copy(hm.at[src_v.at[i % SLAB]], g_v.at[b], sg[b])

        n_my = jnp.minimum(NCHM, E // CM - w * NCHM)

        load_slab(0)
        start(0, 0)

        def pair(j, _):
            for b in (0, 1):
                i = 2 * j + b
                nxt = i + 1

                @pl.when((i >= 1) & (nxt < n_my))
                def _():
                    pltpu.make_async_copy(em.at[pl.ds(0, CM)],
                                          g_v.at[1 - b], ss[1 - b]).wait()

                @pl.when((nxt < n_my) & (nxt % SLAB != 0))
                def _():
                    start(nxt, 1 - b)

                pltpu.make_async_copy(em.at[pl.ds(0, CM)], em_v.at[b],
                                      se[b]).wait()
                pltpu.make_async_copy(hm.at[pl.ds(0, CM)], g_v.at[b],
                                      sg[b]).wait()

                _relu_rows(g_v.at[b], g_v.at[b], em_v.at[b], rows=CM)
                pltpu.async_copy(g_v.at[b], agg.at[dst_v.at[i % SLAB]],
                                 ss[b], add=True)

                @pl.when((nxt < n_my) & (nxt % SLAB == 0))
                def _():
                    load_slab(nxt)
                    start(nxt, 1 - b)
            return 0

        lax.fori_loop(0, n_my // 2, pair, 0)
        pltpu.make_async_copy(em.at[pl.ds(0, CM)], g_v.at[0], ss[0]).wait()
        pltpu.make_async_copy(em.at[pl.ds(0, CM)], g_v.at[1], ss[1]).wait()
        plsc.subcore_barrier()
        pltpu.sync_copy(agg.at[pl.ds(s * ZR, ZR)],
                        out.at[c, pl.ds(s * ZR, ZR)])

    return _sc_message


@functools.cache
def _sc_edge_update_fn():
    @functools.partial(
        pl.kernel,
        out_type=jax.ShapeDtypeStruct((E_PAD, D), jnp.float32),
        mesh=_sc_mesh(),
        scratch_types=[
            pltpu.VMEM((NCH, C), jnp.int32),
            pltpu.VMEM((NCH, C), jnp.int32),
            pltpu.VMEM((2, C, D), jnp.float32),
            pltpu.VMEM((2, C, D), jnp.float32),
            pltpu.VMEM((2, C, D), jnp.float32),
            pltpu.SemaphoreType.DMA,
            pltpu.SemaphoreType.DMA,
            pltpu.SemaphoreType.DMA,
            pltpu.SemaphoreType.DMA,
            pltpu.SemaphoreType.DMA,
            pltpu.SemaphoreType.DMA,
            pltpu.SemaphoreType.DMA,
            pltpu.SemaphoreType.DMA,
        ],
    )
    def _sc_edge_update(hu1, hu2, eu, srcp, dstp, out, src_v, dst_v, g1, g2,
                        ev, sv0, sv1, sa0, sa1, sb0, sb1, st0, st1):
        w = _worker_id()
        sv = (sv0, sv1)
        sa = (sa0, sa1)
        sb = (sb0, sb1)
        st = (st0, st1)
        pltpu.sync_copy(srcp.at[w], src_v)
        pltpu.sync_copy(dstp.at[w], dst_v)
        n_my = jnp.minimum(NCH, E // C - w * NCH)

        def start(i, b):
            pltpu.async_copy(eu.at[pl.ds(w * EPW + i * C, C)], ev.at[b],
                             sv[b])
            pltpu.async_copy(hu1.at[src_v.at[i]], g1.at[b], sa[b])
            pltpu.async_copy(hu2.at[dst_v.at[i]], g2.at[b], sb[b])

        start(0, 0)

        def pair(j, _):
            for b in (0, 1):
                i = 2 * j + b
                nxt = i + 1

                @pl.when((i >= 1) & (nxt < n_my))
                def _():
                    pltpu.make_async_copy(eu.at[pl.ds(0, C)], ev.at[1 - b],
                                          st[1 - b]).wait()

                @pl.when(nxt < n_my)
                def _():
                    start(nxt, 1 - b)

                pltpu.make_async_copy(eu.at[pl.ds(0, C)], ev.at[b],
                                      sv[b]).wait()
                pltpu.make_async_copy(hu1.at[pl.ds(0, C)], g1.at[b],
                                      sa[b]).wait()
                pltpu.make_async_copy(hu2.at[pl.ds(0, C)], g2.at[b],
                                      sb[b]).wait()
                _relu_rows(ev.at[b], g1.at[b], g2.at[b], ev.at[b], rows=C)
                pltpu.async_copy(ev.at[b], out.at[pl.ds(w * EPW + i * C, C)],
                                 st[b])
            return 0

        lax.fori_loop(0, n_my // 2, pair, 0)
        pltpu.make_async_copy(eu.at[pl.ds(0, C)], ev.at[0], st[0]).wait()
        pltpu.make_async_copy(eu.at[pl.ds(0, C)], ev.at[1], st[1]).wait()

    return _sc_edge_update


_APW = N_ACTIVE // NW


@functools.cache
def _sc_active_gather_fn():
    @functools.partial(
        pl.kernel,
        out_type=jax.ShapeDtypeStruct((N_ACTIVE, D), jnp.float32),
        mesh=_sc_mesh(),
        scratch_types=[
            pltpu.VMEM((_APW,), jnp.int32),
            pltpu.VMEM((_APW, D), jnp.float32),
            pltpu.SemaphoreType.DMA,
        ],
    )
    def _sc_active_gather(h, nid, out, idx_v, rows_v, sem):
        w = _worker_id()
        base = w * _APW
        pltpu.sync_copy(nid.at[pl.ds(base, _APW)], idx_v)
        pltpu.async_copy(h.at[idx_v], rows_v, sem).wait()
        pltpu.sync_copy(rows_v, out.at[pl.ds(base, _APW)])

    return _sc_active_gather


def _dot(a, b):
    return jnp.dot(a, b, preferred_element_type=jnp.float32)


def _encode_nodes_body(x, wne, bne, wmh, wself, bn, hm_o, hs_o):
    h = jnp.maximum(_dot(x[...], wne[...]) + bne[...], 0.0)
    hm_o[...] = _dot(h, wmh[...])
    hs_o[...] = _dot(h, wself[...]) + bn[...]


def _encode_nodes(x, wne, bne, wmh, wself, bn):
    return pl.pallas_call(
        _encode_nodes_body,
        out_shape=[jax.ShapeDtypeStruct((N, D), jnp.float32)] * 2,
    )(x, wne, bne, wmh, wself, bn)


_BE = 4000
_GE = E // _BE


def _encode_edges_body(ea, wee, bee, wme, e_o, em_o):
    e = jnp.maximum(_dot(ea[...], wee[...]) + bee[...], 0.0)
    e_o[...] = e
    em_o[...] = _dot(e, wme[...])


def _encode_edges(ea, wee, bee, wme):
    blk = lambda r, c: pl.BlockSpec((r, c), lambda i: (0, 0))
    return pl.pallas_call(
        _encode_edges_body,
        grid=(_GE,),
        in_specs=[pl.BlockSpec((_BE, EDGE_DIM), lambda i: (i, 0)),
                  blk(EDGE_DIM, D), blk(1, D), blk(D, D)],
        out_specs=[pl.BlockSpec((_BE, D), lambda i: (i, 0))] * 2,
        out_shape=[jax.ShapeDtypeStruct((E_PAD, D), jnp.float32)] * 2,
    )(ea, wee, bee, wme)


def _edge_mm1_body(e, wme, em_o):
    em_o[...] = _dot(e[...], wme[...])


def _edge_mm1(e, wme):
    return pl.pallas_call(
        _edge_mm1_body,
        grid=(_GE,),
        in_specs=[pl.BlockSpec((_BE, D), lambda i: (i, 0)),
                  pl.BlockSpec((D, D), lambda i: (0, 0))],
        out_specs=pl.BlockSpec((_BE, D), lambda i: (i, 0)),
        out_shape=jax.ShapeDtypeStruct((E_PAD, D), jnp.float32),
    )(e, wme)


def _eu_mm_body(e, wue, be, eu_o):
    eu_o[...] = _dot(e[...], wue[...]) + be[...]


def _eu_mm(e, wue, be):
    blk = lambda r, c: pl.BlockSpec((r, c), lambda i: (0, 0))
    return pl.pallas_call(
        _eu_mm_body,
        grid=(_GE,),
        in_specs=[pl.BlockSpec((_BE, D), lambda i: (i, 0)),
                  blk(D, D), blk(1, D)],
        out_specs=pl.BlockSpec((_BE, D), lambda i: (i, 0)),
        out_shape=jax.ShapeDtypeStruct((E_PAD, D), jnp.float32),
    )(e, wue, be)


def _node_update_mid_body(hs, agg, wmh, wself, bn, wes, wed,
                          hm_o, hs_o, hu1_o, hu2_o):
    a = agg[...]
    h = jnp.maximum(hs[...] + a[0, :N, :] + a[1, :N, :], 0.0)
    hm_o[...] = _dot(h, wmh[...])
    hs_o[...] = _dot(h, wself[...]) + bn[...]
    hu1_o[...] = _dot(h, wes[...])
    hu2_o[...] = _dot(h, wed[...])


def _node_update_mid(hs, agg, wmh, wself, bn, wes, wed):
    return pl.pallas_call(
        _node_update_mid_body,
        out_shape=[jax.ShapeDtypeStruct((N, D), jnp.float32)] * 4,
    )(hs, agg, wmh, wself, bn, wes, wed)


def _node_update_last_body(hs, agg, h_o):
    a = agg[...]
    h_o[...] = jnp.maximum(hs[...] + a[0, :N, :] + a[1, :N, :], 0.0)


def _node_update_last(hs, agg):
    return pl.pallas_call(
        _node_update_last_body,
        out_shape=jax.ShapeDtypeStruct((N, D), jnp.float32),
    )(hs, agg)


def _mlp_body(ha, w1, b1, w2, b2, out):
    z = jnp.maximum(_dot(ha[...], w1[...]) + b1[...], 0.0)
    out[...] = _dot(z, w2[...]) + b2[...]


def _mlp(ha, w1, b1, w2, b2):
    return pl.pallas_call(
        _mlp_body,
        out_shape=jax.ShapeDtypeStruct((N_ACTIVE, 1), jnp.float32),
    )(ha, w1, b1, w2, b2)


def kernel(x, edge_attr, edge_index, active_nid, W_ne, b_ne, W_ee, b_ee,
           W_msg, W_self, b_n, W_eu, b_e, W1, b1, W2, b2):
    f32 = jnp.float32
    src = edge_index[0].astype(jnp.int32)
    dst = edge_index[1].astype(jnp.int32)
    npad = E_PAD - E
    pad_i = jnp.arange(npad, dtype=jnp.int32)
    src_p = jnp.concatenate([src, (pad_i * 97) % N])
    dst_p = jnp.concatenate([dst, N + pad_i % (NAGG - N)])
    src_pm = src_p.reshape(NW, NCHM, CM)
    dst_pm = dst_p.reshape(NW, NCHM, CM)
    src_pe = src_p.reshape(NW, NCH, C)
    dst_pe = dst_p.reshape(NW, NCH, C)
    zeros_agg = jnp.zeros((NAGG, D), f32)

    Wm_h, Wm_e = W_msg[:D], W_msg[D:]
    We_s, We_d, We_e = W_eu[:D], W_eu[D:2 * D], W_eu[2 * D:]
    row = lambda b: b.reshape(1, -1)

    hm, hs = _encode_nodes(x, W_ne, row(b_ne), Wm_h, W_self, row(b_n))
    e, em = _encode_edges(edge_attr, W_ee, row(b_ee), Wm_e)

    for layer in range(3):
        agg = _sc_message_fn()(hm, em, src_pm, dst_pm, zeros_agg)
        if layer < 2:
            eu = _eu_mm(e, We_e, row(b_e))
            hm, hs, hu1, hu2 = _node_update_mid(
                hs, agg, Wm_h, W_self, row(b_n), We_s, We_d)
            e = _sc_edge_update_fn()(hu1, hu2, eu, src_pe, dst_pe)
            em = _edge_mm1(e, Wm_e)
        else:
            h_fin = _node_update_last(hs, agg)

    ha = _sc_active_gather_fn()(h_fin, active_nid)
    logits = _mlp(ha, W1, row(b1), W2, b2.reshape(1, 1))
    return (logits, active_nid)


# speedup vs baseline: 1.0226x; 1.0174x over previous
import functools

import jax
import jax.numpy as jnp
from jax import lax
from jax.experimental import pallas as pl
from jax.experimental.pallas import tpu as pltpu
from jax.experimental.pallas import tpu_sc as plsc

N = 10000
E = 320000
D = 128
EDGE_DIM = 16
N_ACTIVE = 4096

NC, NS = 2, 16
NW = NC * NS
C = 128
E_PAD = 327680
EPW = E_PAD // NW
NCH = EPW // C
NAGG = 10112
ZR = NAGG // NS

def _sc_mesh():
    return plsc.VectorSubcoreMesh(
        core_axis_name="c", subcore_axis_name="s",
        num_cores=NC, num_subcores=NS)


def _worker_id():
    return lax.axis_index("s") * NC + lax.axis_index("c")


def _relu_rows(dst_ref, *src_refs, rows):
    def row(r, _):
        for cg in range(D // 16):
            sl = pl.ds(cg * 16, 16)
            acc = src_refs[0][r, sl]
            for s in src_refs[1:]:
                acc = acc + s[r, sl]
            dst_ref[r, sl] = jnp.maximum(acc, 0.0)
        return 0
    lax.fori_loop(0, rows, row, 0)


CM = 80
NCHM = EPW // CM
SLAB = 16


@functools.cache
def _sc_message_fn(half):
    @functools.partial(
        pl.kernel,
        out_type=jax.ShapeDtypeStruct((NC, NAGG, D), jnp.float32),
        mesh=_sc_mesh(),
        scratch_types=[
            pltpu.VMEM((SLAB, CM), jnp.int32),
            pltpu.VMEM((SLAB, CM), jnp.int32),
            pltpu.VMEM((2, CM, D), jnp.float32),
            pltpu.VMEM((2, CM, D), jnp.float32),
            pltpu.VMEM_SHARED((NAGG, D), jnp.float32),
            pltpu.SemaphoreType.DMA,
            pltpu.SemaphoreType.DMA,
            pltpu.SemaphoreType.DMA,
            pltpu.SemaphoreType.DMA,
            pltpu.SemaphoreType.DMA,
            pltpu.SemaphoreType.DMA,
        ],
    )
    def _sc_message(hm, em, srcp, dstp, zinit, out, src_v, dst_v, em_v, g_v,
                    agg, se0, se1, sg0, sg1, ss0, ss1):
        c = lax.axis_index("c")
        s = lax.axis_index("s")
        w = _worker_id()
        se = (se0, se1)
        sg = (sg0, sg1)
        ss = (ss0, ss1)
        pltpu.sync_copy(zinit.at[c, pl.ds(s * ZR, ZR)],
                        agg.at[pl.ds(s * ZR, ZR)])
        plsc.subcore_barrier()

        if half is None:
            chunk0 = w * NCHM
            n_my = jnp.minimum(NCHM, E // CM - chunk0)
        else:
            chunk0 = half * 2000 + w * 64
            n_my = jnp.clip(2000 - w * 64, 0, 64)

        def load_slab(i):
            i = chunk0 + pl.multiple_of(i, SLAB)
            pltpu.sync_copy(srcp.at[pl.ds(i, SLAB)], src_v)
            pltpu.sync_copy(dstp.at[pl.ds(i, SLAB)], dst_v)

        def start(i, b):
            pltpu.async_copy(em.at[pl.ds((chunk0 + i) * CM, CM)], em_v.at[b],
                             se[b])
            pltpu.async_copy(hm.at[src_v.at[i % SLAB]], g_v.at[b], sg[b])

        @pl.when(n_my > 0)
        def _():
            load_slab(0)
            start(0, 0)

        def pair(j, _):
            for b in (0, 1):
                i = 2 * j + b
                nxt = i + 1

                @pl.when((i >= 1) & (nxt < n_my))
                def _():
                    pltpu.make_async_copy(em.at[pl.ds(0, CM)],
                                          g_v.at[1 - b], ss[1 - b]).wait()

                @pl.when((nxt < n_my) & (nxt % SLAB != 0))
                def _():
                    start(nxt, 1 - b)

                pltpu.make_async_copy(em.at[pl.ds(0, CM)], em_v.at[b],
                                      se[b]).wait()
                pltpu.make_async_copy(hm.at[pl.ds(0, CM)], g_v.at[b],
                                      sg[b]).wait()

                _relu_rows(g_v.at[b], g_v.at[b], em_v.at[b], rows=CM)
                pltpu.async_copy(g_v.at[b], agg.at[dst_v.at[i % SLAB]],
                                 ss[b], add=True)

                @pl.when((nxt < n_my) & (nxt % SLAB == 0))
                def _():
                    load_slab(nxt)
                    start(nxt, 1 - b)
            return 0

        lax.fori_loop(0, n_my // 2, pair, 0)

        @pl.when(n_my > 0)
        def _():
            pltpu.make_async_copy(em.at[pl.ds(0, CM)], g_v.at[0],
                                  ss[0]).wait()
            pltpu.make_async_copy(em.at[pl.ds(0, CM)], g_v.at[1],
                                  ss[1]).wait()

        plsc.subcore_barrier()
        pltpu.sync_copy(agg.at[pl.ds(s * ZR, ZR)],
                        out.at[c, pl.ds(s * ZR, ZR)])

    return _sc_message


@functools.cache
def _sc_edge_update_fn():
    @functools.partial(
        pl.kernel,
        out_type=jax.ShapeDtypeStruct((E_PAD, D), jnp.float32),
        mesh=_sc_mesh(),
        scratch_types=[
            pltpu.VMEM((NCH, C), jnp.int32),
            pltpu.VMEM((NCH, C), jnp.int32),
            pltpu.VMEM((2, C, D), jnp.float32),
            pltpu.VMEM((2, C, D), jnp.float32),
            pltpu.VMEM((2, C, D), jnp.float32),
            pltpu.SemaphoreType.DMA,
            pltpu.SemaphoreType.DMA,
            pltpu.SemaphoreType.DMA,
            pltpu.SemaphoreType.DMA,
            pltpu.SemaphoreType.DMA,
            pltpu.SemaphoreType.DMA,
            pltpu.SemaphoreType.DMA,
            pltpu.SemaphoreType.DMA,
        ],
    )
    def _sc_edge_update(hu1, hu2, eu, srcp, dstp, out, src_v, dst_v, g1, g2,
                        ev, sv0, sv1, sa0, sa1, sb0, sb1, st0, st1):
        w = _worker_id()
        sv = (sv0, sv1)
        sa = (sa0, sa1)
        sb = (sb0, sb1)
        st = (st0, st1)
        pltpu.sync_copy(srcp.at[w], src_v)
        pltpu.sync_copy(dstp.at[w], dst_v)
        n_my = jnp.minimum(NCH, E // C - w * NCH)

        def start(i, b):
            pltpu.async_copy(eu.at[pl.ds(w * EPW + i * C, C)], ev.at[b],
                             sv[b])
            pltpu.async_copy(hu1.at[src_v.at[i]], g1.at[b], sa[b])
            pltpu.async_copy(hu2.at[dst_v.at[i]], g2.at[b], sb[b])

        start(0, 0)

        def pair(j, _):
            for b in (0, 1):
                i = 2 * j + b
                nxt = i + 1

                @pl.when((i >= 1) & (nxt < n_my))
                def _():
                    pltpu.make_async_copy(eu.at[pl.ds(0, C)], ev.at[1 - b],
                                          st[1 - b]).wait()

                @pl.when(nxt < n_my)
                def _():
                    start(nxt, 1 - b)

                pltpu.make_async_copy(eu.at[pl.ds(0, C)], ev.at[b],
                                      sv[b]).wait()
                pltpu.make_async_copy(hu1.at[pl.ds(0, C)], g1.at[b],
                                      sa[b]).wait()
                pltpu.make_async_copy(hu2.at[pl.ds(0, C)], g2.at[b],
                                      sb[b]).wait()
                _relu_rows(ev.at[b], g1.at[b], g2.at[b], ev.at[b], rows=C)
                pltpu.async_copy(ev.at[b], out.at[pl.ds(w * EPW + i * C, C)],
                                 st[b])
            return 0

        lax.fori_loop(0, n_my // 2, pair, 0)
        pltpu.make_async_copy(eu.at[pl.ds(0, C)], ev.at[0], st[0]).wait()
        pltpu.make_async_copy(eu.at[pl.ds(0, C)], ev.at[1], st[1]).wait()

    return _sc_edge_update


_APW = N_ACTIVE // NW


@functools.cache
def _sc_active_gather_fn():
    @functools.partial(
        pl.kernel,
        out_type=jax.ShapeDtypeStruct((N_ACTIVE, D), jnp.float32),
        mesh=_sc_mesh(),
        scratch_types=[
            pltpu.VMEM((_APW,), jnp.int32),
            pltpu.VMEM((_APW, D), jnp.float32),
            pltpu.SemaphoreType.DMA,
        ],
    )
    def _sc_active_gather(h, nid, out, idx_v, rows_v, sem):
        w = _worker_id()
        base = w * _APW
        pltpu.sync_copy(nid.at[pl.ds(base, _APW)], idx_v)
        pltpu.async_copy(h.at[idx_v], rows_v, sem).wait()
        pltpu.sync_copy(rows_v, out.at[pl.ds(base, _APW)])

    return _sc_active_gather


def _dot(a, b):
    return jnp.dot(a, b, preferred_element_type=jnp.float32)


def _encode_nodes_body(x, wne, bne, wmh, wself, bn, hm_o, hs_o):
    h = jnp.maximum(_dot(x[...], wne[...]) + bne[...], 0.0)
    hm_o[...] = _dot(h, wmh[...])
    hs_o[...] = _dot(h, wself[...]) + bn[...]


def _encode_nodes(x, wne, bne, wmh, wself, bn):
    return pl.pallas_call(
        _encode_nodes_body,
        out_shape=[jax.ShapeDtypeStruct((N, D), jnp.float32)] * 2,
    )(x, wne, bne, wmh, wself, bn)


_BE = 4000
_GE = E // _BE


def _encode_edges_body(ea, wee, bee, wme, e_o, em_o):
    e = jnp.maximum(_dot(ea[...], wee[...]) + bee[...], 0.0)
    e_o[...] = e
    em_o[...] = _dot(e, wme[...])


def _encode_edges(ea, wee, bee, wme):
    blk = lambda r, c: pl.BlockSpec((r, c), lambda i: (0, 0))
    return pl.pallas_call(
        _encode_edges_body,
        grid=(_GE,),
        in_specs=[pl.BlockSpec((_BE, EDGE_DIM), lambda i: (i, 0)),
                  blk(EDGE_DIM, D), blk(1, D), blk(D, D)],
        out_specs=[pl.BlockSpec((_BE, D), lambda i: (i, 0))] * 2,
        out_shape=[jax.ShapeDtypeStruct((E_PAD, D), jnp.float32)] * 2,
    )(ea, wee, bee, wme)


def _edge_mm1_body(e, wme, em_o):
    em_o[...] = _dot(e[...], wme[...])


def _edge_mm1(e, wme, half=None):
    off = 0 if half is None else half * (_GE // 2)
    g = _GE if half is None else _GE // 2
    return pl.pallas_call(
        _edge_mm1_body,
        grid=(g,),
        in_specs=[pl.BlockSpec((_BE, D), lambda i: (i + off, 0)),
                  pl.BlockSpec((D, D), lambda i: (0, 0))],
        out_specs=pl.BlockSpec((_BE, D), lambda i: (i + off, 0)),
        out_shape=jax.ShapeDtypeStruct((E_PAD, D), jnp.float32),
    )(e, wme)


def _eu_mm_body(e, wue, be, eu_o):
    eu_o[...] = _dot(e[...], wue[...]) + be[...]


def _eu_mm(e, wue, be):
    blk = lambda r, c: pl.BlockSpec((r, c), lambda i: (0, 0))
    return pl.pallas_call(
        _eu_mm_body,
        grid=(_GE,),
        in_specs=[pl.BlockSpec((_BE, D), lambda i: (i, 0)),
                  blk(D, D), blk(1, D)],
        out_specs=pl.BlockSpec((_BE, D), lambda i: (i, 0)),
        out_shape=jax.ShapeDtypeStruct((E_PAD, D), jnp.float32),
    )(e, wue, be)


def _node_update_mid_body(hs, agg, wmh, wself, bn, wes, wed,
                          hm_o, hs_o, hu1_o, hu2_o):
    a = agg[...]
    h = jnp.maximum(hs[...] + a[0, :N, :] + a[1, :N, :], 0.0)
    hm_o[...] = _dot(h, wmh[...])
    hs_o[...] = _dot(h, wself[...]) + bn[...]
    hu1_o[...] = _dot(h, wes[...])
    hu2_o[...] = _dot(h, wed[...])


def _node_update_mid(hs, agg, wmh, wself, bn, wes, wed):
    return pl.pallas_call(
        _node_update_mid_body,
        out_shape=[jax.ShapeDtypeStruct((N, D), jnp.float32)] * 4,
    )(hs, agg, wmh, wself, bn, wes, wed)


def _node_update_last_body(hs, agg, h_o):
    a = agg[...]
    h_o[...] = jnp.maximum(hs[...] + a[0, :N, :] + a[1, :N, :], 0.0)


def _node_update_last(hs, agg):
    return pl.pallas_call(
        _node_update_last_body,
        out_shape=jax.ShapeDtypeStruct((N, D), jnp.float32),
    )(hs, agg)


def _mlp_body(ha, w1, b1, w2, b2, out):
    z = jnp.maximum(_dot(ha[...], w1[...]) + b1[...], 0.0)
    out[...] = _dot(z, w2[...]) + b2[...]


def _mlp(ha, w1, b1, w2, b2):
    return pl.pallas_call(
        _mlp_body,
        out_shape=jax.ShapeDtypeStruct((N_ACTIVE, 1), jnp.float32),
    )(ha, w1, b1, w2, b2)


def kernel(x, edge_attr, edge_index, active_nid, W_ne, b_ne, W_ee, b_ee,
           W_msg, W_self, b_n, W_eu, b_e, W1, b1, W2, b2):
    f32 = jnp.float32
    src = edge_index[0].astype(jnp.int32)
    dst = edge_index[1].astype(jnp.int32)
    npad = E_PAD - E
    pad_i = jnp.arange(npad, dtype=jnp.int32)
    src_p = jnp.concatenate([src, (pad_i * 97) % N])
    dst_p = jnp.concatenate([dst, N + pad_i % (NAGG - N)])
    src_pm = src_p.reshape(E_PAD // CM, CM)
    dst_pm = dst_p.reshape(E_PAD // CM, CM)
    src_pe = src_p.reshape(NW, NCH, C)
    dst_pe = dst_p.reshape(NW, NCH, C)
    zeros_agg = jnp.zeros((NC, NAGG, D), f32)

    Wm_h, Wm_e = W_msg[:D], W_msg[D:]
    We_s, We_d, We_e = W_eu[:D], W_eu[D:2 * D], W_eu[2 * D:]
    row = lambda b: b.reshape(1, -1)

    hm, hs = _encode_nodes(x, W_ne, row(b_ne), Wm_h, W_self, row(b_n))
    e, em = _encode_edges(edge_attr, W_ee, row(b_ee), Wm_e)

    for layer in range(3):
        if layer == 0:
            agg = _sc_message_fn(None)(hm, em, src_pm, dst_pm, zeros_agg)
        else:
            agg_a = _sc_message_fn(0)(hm, em_a, src_pm, dst_pm, zeros_agg)
            agg = _sc_message_fn(1)(hm, em_b, src_pm, dst_pm, agg_a)
        if layer < 2:
            eu = _eu_mm(e, We_e, row(b_e))
            hm, hs, hu1, hu2 = _node_update_mid(
                hs, agg, Wm_h, W_self, row(b_n), We_s, We_d)
            e = _sc_edge_update_fn()(hu1, hu2, eu, src_pe, dst_pe)
            em_a = _edge_mm1(e, Wm_e, 0)
            em_b = _edge_mm1(e, Wm_e, 1)
        else:
            h_fin = _node_update_last(hs, agg)

    ha = _sc_active_gather_fn()(h_fin, active_nid)
    logits = _mlp(ha, W1, row(b1), W2, b2.reshape(1, 1))
    return (logits, active_nid)


# speedup vs baseline: 1.0236x; 1.0010x over previous
import functools

import jax
import jax.numpy as jnp
from jax import lax
from jax.experimental import pallas as pl
from jax.experimental.pallas import tpu as pltpu
from jax.experimental.pallas import tpu_sc as plsc

N = 10000
E = 320000
D = 128
EDGE_DIM = 16
N_ACTIVE = 4096

NC, NS = 2, 16
NW = NC * NS
C = 128
E_PAD = 327680
EPW = E_PAD // NW
NCH = EPW // C
NAGG = 10112
ZR = NAGG // NS

def _sc_mesh():
    return plsc.VectorSubcoreMesh(
        core_axis_name="c", subcore_axis_name="s",
        num_cores=NC, num_subcores=NS)


def _worker_id():
    return lax.axis_index("s") * NC + lax.axis_index("c")


def _relu_rows(dst_ref, *src_refs, rows):
    def row(r, _):
        for cg in range(D // 16):
            sl = pl.ds(cg * 16, 16)
            acc = src_refs[0][r, sl]
            for s in src_refs[1:]:
                acc = acc + s[r, sl]
            dst_ref[r, sl] = jnp.maximum(acc, 0.0)
        return 0
    lax.fori_loop(0, rows, row, 0)


CM = 80
NCHM = EPW // CM
SLAB = 16


@functools.cache
def _sc_message_fn(half):
    @functools.partial(
        pl.kernel,
        out_type=jax.ShapeDtypeStruct((NC, NAGG, D), jnp.float32),
        mesh=_sc_mesh(),
        scratch_types=[
            pltpu.VMEM((SLAB * CM,), jnp.int32),
            pltpu.VMEM((SLAB, CM), jnp.int32),
            pltpu.VMEM((2, CM, D), jnp.float32),
            pltpu.VMEM((2, CM, D), jnp.float32),
            pltpu.VMEM_SHARED((NAGG, D), jnp.float32),
            pltpu.SemaphoreType.DMA,
            pltpu.SemaphoreType.DMA,
            pltpu.SemaphoreType.DMA,
            pltpu.SemaphoreType.DMA,
            pltpu.SemaphoreType.DMA,
            pltpu.SemaphoreType.DMA,
        ],
    )
    def _sc_message(hm, em, srcp, dstp, zinit, out, src_v, dst_v, em_v, g_v,
                    agg, se0, se1, sg0, sg1, ss0, ss1):
        c = lax.axis_index("c")
        s = lax.axis_index("s")
        w = _worker_id()
        se = (se0, se1)
        sg = (sg0, sg1)
        ss = (ss0, ss1)
        pltpu.sync_copy(zinit.at[c, pl.ds(s * ZR, ZR)],
                        agg.at[pl.ds(s * ZR, ZR)])
        plsc.subcore_barrier()

        if half is None:
            chunk0 = w * NCHM
            n_my = jnp.minimum(NCHM, E // CM - chunk0)
        else:
            chunk0 = half * 2000 + w * 64
            n_my = jnp.clip(2000 - w * 64, 0, 64)

        def load_slab(i):
            i = chunk0 + pl.multiple_of(i, SLAB)
            pltpu.sync_copy(srcp.at[pl.ds(i * CM, SLAB * CM)], src_v)
            pltpu.sync_copy(dstp.at[pl.ds(i, SLAB)], dst_v)

        def start(i, b):
            pltpu.async_copy(em.at[pl.ds((chunk0 + i) * CM, CM)], em_v.at[b],
                             se[b])
            pltpu.async_copy(hm.at[src_v.at[pl.ds((i % SLAB) * CM, CM)]],
                             g_v.at[b], sg[b])

        @pl.when(n_my > 0)
        def _():
            load_slab(0)
            start(0, 0)

        def pair(j, _):
            for b in (0, 1):
                i = 2 * j + b
                nxt = i + 1

                @pl.when((i >= 1) & (nxt < n_my))
                def _():
                    pltpu.make_async_copy(em.at[pl.ds(0, CM)],
                                          g_v.at[1 - b], ss[1 - b]).wait()

                @pl.when((nxt < n_my) & (nxt % SLAB != 0))
                def _():
                    start(nxt, 1 - b)

                pltpu.make_async_copy(em.at[pl.ds(0, CM)], em_v.at[b],
                                      se[b]).wait()
                pltpu.make_async_copy(hm.at[pl.ds(0, CM)], g_v.at[b],
                                      sg[b]).wait()

                _relu_rows(g_v.at[b], g_v.at[b], em_v.at[b], rows=CM)
                pltpu.async_copy(g_v.at[b], agg.at[dst_v.at[i % SLAB]],
                                 ss[b], add=True)

                @pl.when((nxt < n_my) & (nxt % SLAB == 0))
                def _():
                    load_slab(nxt)
                    start(nxt, 1 - b)
            return 0

        lax.fori_loop(0, n_my // 2, pair, 0)

        @pl.when(n_my > 0)
        def _():
            pltpu.make_async_copy(em.at[pl.ds(0, CM)], g_v.at[0],
                                  ss[0]).wait()
            pltpu.make_async_copy(em.at[pl.ds(0, CM)], g_v.at[1],
                                  ss[1]).wait()

        plsc.subcore_barrier()
        pltpu.sync_copy(agg.at[pl.ds(s * ZR, ZR)],
                        out.at[c, pl.ds(s * ZR, ZR)])

    return _sc_message


@functools.cache
def _sc_edge_update_fn():
    @functools.partial(
        pl.kernel,
        out_type=jax.ShapeDtypeStruct((E_PAD, D), jnp.float32),
        mesh=_sc_mesh(),
        scratch_types=[
            pltpu.VMEM((EPW,), jnp.int32),
            pltpu.VMEM((EPW,), jnp.int32),
            pltpu.VMEM((2, C, D), jnp.float32),
            pltpu.VMEM((2, C, D), jnp.float32),
            pltpu.VMEM((2, C, D), jnp.float32),
            pltpu.SemaphoreType.DMA,
            pltpu.SemaphoreType.DMA,
            pltpu.SemaphoreType.DMA,
            pltpu.SemaphoreType.DMA,
            pltpu.SemaphoreType.DMA,
            pltpu.SemaphoreType.DMA,
            pltpu.SemaphoreType.DMA,
            pltpu.SemaphoreType.DMA,
        ],
    )
    def _sc_edge_update(hu1, hu2, eu, srcp, dstp, out, src_v, dst_v, g1, g2,
                        ev, sv0, sv1, sa0, sa1, sb0, sb1, st0, st1):
        w = _worker_id()
        sv = (sv0, sv1)
        sa = (sa0, sa1)
        sb = (sb0, sb1)
        st = (st0, st1)
        pltpu.sync_copy(srcp.at[pl.ds(w * EPW, EPW)], src_v)
        pltpu.sync_copy(dstp.at[pl.ds(w * EPW, EPW)], dst_v)
        n_my = jnp.minimum(NCH, E // C - w * NCH)

        def start(i, b):
            pltpu.async_copy(eu.at[pl.ds(w * EPW + i * C, C)], ev.at[b],
                             sv[b])
            pltpu.async_copy(hu1.at[src_v.at[pl.ds(i * C, C)]], g1.at[b],
                             sa[b])
            pltpu.async_copy(hu2.at[dst_v.at[pl.ds(i * C, C)]], g2.at[b],
                             sb[b])

        start(0, 0)

        def pair(j, _):
            for b in (0, 1):
                i = 2 * j + b
                nxt = i + 1

                @pl.when((i >= 1) & (nxt < n_my))
                def _():
                    pltpu.make_async_copy(eu.at[pl.ds(0, C)], ev.at[1 - b],
                                          st[1 - b]).wait()

                @pl.when(nxt < n_my)
                def _():
                    start(nxt, 1 - b)

                pltpu.make_async_copy(eu.at[pl.ds(0, C)], ev.at[b],
                                      sv[b]).wait()
                pltpu.make_async_copy(hu1.at[pl.ds(0, C)], g1.at[b],
                                      sa[b]).wait()
                pltpu.make_async_copy(hu2.at[pl.ds(0, C)], g2.at[b],
                                      sb[b]).wait()
                _relu_rows(ev.at[b], g1.at[b], g2.at[b], ev.at[b], rows=C)
                pltpu.async_copy(ev.at[b], out.at[pl.ds(w * EPW + i * C, C)],
                                 st[b])
            return 0

        lax.fori_loop(0, n_my // 2, pair, 0)
        pltpu.make_async_copy(eu.at[pl.ds(0, C)], ev.at[0], st[0]).wait()
        pltpu.make_async_copy(eu.at[pl.ds(0, C)], ev.at[1], st[1]).wait()

    return _sc_edge_update


_APW = N_ACTIVE // NW


@functools.cache
def _sc_active_gather_fn():
    @functools.partial(
        pl.kernel,
        out_type=jax.ShapeDtypeStruct((N_ACTIVE, D), jnp.float32),
        mesh=_sc_mesh(),
        scratch_types=[
            pltpu.VMEM((_APW,), jnp.int32),
            pltpu.VMEM((_APW, D), jnp.float32),
            pltpu.SemaphoreType.DMA,
        ],
    )
    def _sc_active_gather(h, nid, out, idx_v, rows_v, sem):
        w = _worker_id()
        base = w * _APW
        pltpu.sync_copy(nid.at[pl.ds(base, _APW)], idx_v)
        pltpu.async_copy(h.at[idx_v], rows_v, sem).wait()
        pltpu.sync_copy(rows_v, out.at[pl.ds(base, _APW)])

    return _sc_active_gather


def _dot(a, b):
    return jnp.dot(a, b, preferred_element_type=jnp.float32)


def _encode_nodes_body(x, wne, bne, wmh, wself, bn, hm_o, hs_o):
    h = jnp.maximum(_dot(x[...], wne[...]) + bne[...], 0.0)
    hm_o[...] = _dot(h, wmh[...])
    hs_o[...] = _dot(h, wself[...]) + bn[...]


def _encode_nodes(x, wne, bne, wmh, wself, bn):
    return pl.pallas_call(
        _encode_nodes_body,
        out_shape=[jax.ShapeDtypeStruct((N, D), jnp.float32)] * 2,
    )(x, wne, bne, wmh, wself, bn)


_BE = 4000
_GE = E // _BE


def _encode_edges_body(ea, wee, bee, wme, e_o, em_o):
    e = jnp.maximum(_dot(ea[...], wee[...]) + bee[...], 0.0)
    e_o[...] = e
    em_o[...] = _dot(e, wme[...])


def _encode_edges(ea, wee, bee, wme):
    blk = lambda r, c: pl.BlockSpec((r, c), lambda i: (0, 0))
    return pl.pallas_call(
        _encode_edges_body,
        grid=(_GE,),
        in_specs=[pl.BlockSpec((_BE, EDGE_DIM), lambda i: (i, 0)),
                  blk(EDGE_DIM, D), blk(1, D), blk(D, D)],
        out_specs=[pl.BlockSpec((_BE, D), lambda i: (i, 0))] * 2,
        out_shape=[jax.ShapeDtypeStruct((E_PAD, D), jnp.float32)] * 2,
    )(ea, wee, bee, wme)


def _edge_mm1_body(e, wme, em_o):
    em_o[...] = _dot(e[...], wme[...])


def _edge_mm1(e, wme, half=None):
    off = 0 if half is None else half * (_GE // 2)
    g = _GE if half is None else _GE // 2
    return pl.pallas_call(
        _edge_mm1_body,
        grid=(g,),
        in_specs=[pl.BlockSpec((_BE, D), lambda i: (i + off, 0)),
                  pl.BlockSpec((D, D), lambda i: (0, 0))],
        out_specs=pl.BlockSpec((_BE, D), lambda i: (i + off, 0)),
        out_shape=jax.ShapeDtypeStruct((E_PAD, D), jnp.float32),
    )(e, wme)


def _eu_mm_body(e, wue, be, eu_o):
    eu_o[...] = _dot(e[...], wue[...]) + be[...]


def _eu_mm(e, wue, be):
    blk = lambda r, c: pl.BlockSpec((r, c), lambda i: (0, 0))
    return pl.pallas_call(
        _eu_mm_body,
        grid=(_GE,),
        in_specs=[pl.BlockSpec((_BE, D), lambda i: (i, 0)),
                  blk(D, D), blk(1, D)],
        out_specs=pl.BlockSpec((_BE, D), lambda i: (i, 0)),
        out_shape=jax.ShapeDtypeStruct((E_PAD, D), jnp.float32),
    )(e, wue, be)


def _node_update_mid_body(hs, agg, wmh, wself, bn, wes, wed,
                          hm_o, hs_o, hu1_o, hu2_o):
    a = agg[...]
    h = jnp.maximum(hs[...] + a[0, :N, :] + a[1, :N, :], 0.0)
    hm_o[...] = _dot(h, wmh[...])
    hs_o[...] = _dot(h, wself[...]) + bn[...]
    hu1_o[...] = _dot(h, wes[...])
    hu2_o[...] = _dot(h, wed[...])


def _node_update_mid(hs, agg, wmh, wself, bn, wes, wed):
    return pl.pallas_call(
        _node_update_mid_body,
        out_shape=[jax.ShapeDtypeStruct((N, D), jnp.float32)] * 4,
    )(hs, agg, wmh, wself, bn, wes, wed)


def _node_update_last_body(hs, agg, h_o):
    a = agg[...]
    h_o[...] = jnp.maximum(hs[...] + a[0, :N, :] + a[1, :N, :], 0.0)


def _node_update_last(hs, agg):
    return pl.pallas_call(
        _node_update_last_body,
        out_shape=jax.ShapeDtypeStruct((N, D), jnp.float32),
    )(hs, agg)


def _mlp_body(ha, w1, b1, w2, b2, out):
    z = jnp.maximum(_dot(ha[...], w1[...]) + b1[...], 0.0)
    out[...] = _dot(z, w2[...]) + b2[...]


def _mlp(ha, w1, b1, w2, b2):
    return pl.pallas_call(
        _mlp_body,
        out_shape=jax.ShapeDtypeStruct((N_ACTIVE, 1), jnp.float32),
    )(ha, w1, b1, w2, b2)


def kernel(x, edge_attr, edge_index, active_nid, W_ne, b_ne, W_ee, b_ee,
           W_msg, W_self, b_n, W_eu, b_e, W1, b1, W2, b2):
    f32 = jnp.float32
    src = edge_index[0].astype(jnp.int32)
    dst = edge_index[1].astype(jnp.int32)
    npad = E_PAD - E
    pad_i = jnp.arange(npad, dtype=jnp.int32)
    src_p = jnp.concatenate([src, (pad_i * 97) % N])
    dst_p = jnp.concatenate([dst, N + pad_i % (NAGG - N)])
    dst_pm = dst_p.reshape(E_PAD // CM, CM)
    zeros_agg = jnp.zeros((NC, NAGG, D), f32)

    Wm_h, Wm_e = W_msg[:D], W_msg[D:]
    We_s, We_d, We_e = W_eu[:D], W_eu[D:2 * D], W_eu[2 * D:]
    row = lambda b: b.reshape(1, -1)

    hm, hs = _encode_nodes(x, W_ne, row(b_ne), Wm_h, W_self, row(b_n))
    e, em = _encode_edges(edge_attr, W_ee, row(b_ee), Wm_e)

    for layer in range(3):
        if layer == 0:
            agg = _sc_message_fn(None)(hm, em, src_p, dst_pm, zeros_agg)
        else:
            agg_a = _sc_message_fn(0)(hm, em_a, src_p, dst_pm, zeros_agg)
            agg = _sc_message_fn(1)(hm, em_b, src_p, dst_pm, agg_a)
        if layer < 2:
            eu = _eu_mm(e, We_e, row(b_e))
            hm, hs, hu1, hu2 = _node_update_mid(
                hs, agg, Wm_h, W_self, row(b_n), We_s, We_d)
            e = _sc_edge_update_fn()(hu1, hu2, eu, src_p, dst_p)
            em_a = _edge_mm1(e, Wm_e, 0)
            em_b = _edge_mm1(e, Wm_e, 1)
        else:
            h_fin = _node_update_last(hs, agg)

    ha = _sc_active_gather_fn()(h_fin, active_nid)
    logits = _mlp(ha, W1, row(b1), W2, b2.reshape(1, 1))
    return (logits, active_nid)
